# baseline (device time: 34650 ns/iter reference)
import functools

import jax
import jax.numpy as jnp
from jax import lax
from jax.experimental import pallas as pl
from jax.experimental.pallas import tpu as pltpu

N_DEV = 4


def kernel(A, B):
    m, k = A.shape
    _, n = B.shape
    m_out = m // N_DEV

    def body(a_ref, b_ref, out_ref, send_ref, recv_ref, send_sems, recv_sems):
        my = lax.axis_index("i")

        barrier_sem = pltpu.get_barrier_semaphore()
        for off in (1, 2, 3):
            pl.semaphore_signal(
                barrier_sem, inc=1,
                device_id=((my + off) % N_DEV,),
                device_id_type=pl.DeviceIdType.MESH,
            )
        pl.semaphore_wait(barrier_sem, 3)

        rdmas = []
        for off in (2, 1, 3):
            tgt = (my + off) % N_DEV
            slot = off - 1
            send_ref[slot] = jnp.dot(
                a_ref[pl.ds(tgt * m_out, m_out), :], b_ref[:, :],
                preferred_element_type=jnp.float32,
            )
            rdma = pltpu.make_async_remote_copy(
                src_ref=send_ref.at[slot],
                dst_ref=recv_ref.at[3 - off],
                send_sem=send_sems.at[slot],
                recv_sem=recv_sems.at[3 - off],
                device_id=(tgt,),
                device_id_type=pl.DeviceIdType.MESH,
            )
            rdma.start()
            rdmas.append(rdma)

        own = jnp.dot(
            a_ref[pl.ds(my * m_out, m_out), :], b_ref[:, :],
            preferred_element_type=jnp.float32,
        )

        for rdma in rdmas:
            rdma.wait()

        out_ref[:, :] = own + recv_ref[0] + recv_ref[1] + recv_ref[2]

        @functools.partial(pl.run_scoped, sem=pltpu.SemaphoreType.REGULAR)
        def _(sem):
            for off in (1, 2, 3):
                pl.semaphore_signal(
                    sem, inc=1,
                    device_id=((my + off) % N_DEV,),
                    device_id_type=pl.DeviceIdType.MESH,
                )
            pl.semaphore_wait(sem, 3)

    return pl.pallas_call(
        body,
        out_shape=jax.ShapeDtypeStruct((m_out, n), jnp.float32),
        in_specs=[
            pl.BlockSpec(memory_space=pltpu.VMEM),
            pl.BlockSpec(memory_space=pltpu.VMEM),
        ],
        out_specs=pl.BlockSpec(memory_space=pltpu.VMEM),
        scratch_shapes=[
            pltpu.VMEM((N_DEV - 1, m_out, n), jnp.float32),
            pltpu.VMEM((N_DEV - 1, m_out, n), jnp.float32),
            pltpu.SemaphoreType.DMA((N_DEV - 1,)),
            pltpu.SemaphoreType.DMA((N_DEV - 1,)),
        ],
        compiler_params=pltpu.CompilerParams(collective_id=0),
    )(A, B)


# device time: 23366 ns/iter; 1.4829x vs baseline; 1.4829x over previous
import functools

import jax
import jax.numpy as jnp
from jax import lax
from jax.experimental import pallas as pl
from jax.experimental.pallas import tpu as pltpu

N_DEV = 4


def kernel(A, B):
    m, k = A.shape
    _, n = B.shape
    m_out = m // N_DEV

    def body(a_ref, b_ref, out_ref, send_ref, recv_ref, send_sems, recv_sems):
        my = lax.axis_index("i")

        barrier_sem = pltpu.get_barrier_semaphore()
        for off in (1, 2, 3):
            pl.semaphore_signal(
                barrier_sem, inc=1,
                device_id=((my + off) % N_DEV,),
                device_id_type=pl.DeviceIdType.MESH,
            )
        pl.semaphore_wait(barrier_sem, 3)

        rdmas = []
        for off in (2, 1, 3):
            tgt = (my + off) % N_DEV
            slot = off - 1
            send_ref[slot] = jnp.dot(
                a_ref[pl.ds(tgt * m_out, m_out), :], b_ref[:, :],
                preferred_element_type=jnp.float32,
            ).astype(jnp.bfloat16)
            rdma = pltpu.make_async_remote_copy(
                src_ref=send_ref.at[slot],
                dst_ref=recv_ref.at[3 - off],
                send_sem=send_sems.at[slot],
                recv_sem=recv_sems.at[3 - off],
                device_id=(tgt,),
                device_id_type=pl.DeviceIdType.MESH,
            )
            rdma.start()
            rdmas.append(rdma)

        own = jnp.dot(
            a_ref[pl.ds(my * m_out, m_out), :], b_ref[:, :],
            preferred_element_type=jnp.float32,
        )

        for rdma in rdmas:
            rdma.wait()

        out_ref[:, :] = own + (
            recv_ref[0].astype(jnp.float32)
            + recv_ref[1].astype(jnp.float32)
            + recv_ref[2].astype(jnp.float32)
        )

        @functools.partial(pl.run_scoped, sem=pltpu.SemaphoreType.REGULAR)
        def _(sem):
            for off in (1, 2, 3):
                pl.semaphore_signal(
                    sem, inc=1,
                    device_id=((my + off) % N_DEV,),
                    device_id_type=pl.DeviceIdType.MESH,
                )
            pl.semaphore_wait(sem, 3)

    return pl.pallas_call(
        body,
        out_shape=jax.ShapeDtypeStruct((m_out, n), jnp.float32),
        in_specs=[
            pl.BlockSpec(memory_space=pltpu.VMEM),
            pl.BlockSpec(memory_space=pltpu.VMEM),
        ],
        out_specs=pl.BlockSpec(memory_space=pltpu.VMEM),
        scratch_shapes=[
            pltpu.VMEM((N_DEV - 1, m_out, n), jnp.bfloat16),
            pltpu.VMEM((N_DEV - 1, m_out, n), jnp.bfloat16),
            pltpu.SemaphoreType.DMA((N_DEV - 1,)),
            pltpu.SemaphoreType.DMA((N_DEV - 1,)),
        ],
        compiler_params=pltpu.CompilerParams(collective_id=0),
    )(A, B)


# device time: 9800 ns/iter; 3.5357x vs baseline; 2.3843x over previous
import functools

import jax
import jax.numpy as jnp
from jax import lax
from jax.experimental import pallas as pl
from jax.experimental.pallas import tpu as pltpu

N_DEV = 4


def kernel(A, B):
    m, k = A.shape
    _, n = B.shape
    m_out = m // N_DEV

    def body(a_ref, b_ref, out_ref, send_ref, recv_ref, a16_ref, b16_ref,
             send_sems, recv_sems):
        my = lax.axis_index("i")

        barrier_sem = pltpu.get_barrier_semaphore()
        for off in (1, 2, 3):
            pl.semaphore_signal(
                barrier_sem, inc=1,
                device_id=((my + off) % N_DEV,),
                device_id_type=pl.DeviceIdType.MESH,
            )
        pl.semaphore_wait(barrier_sem, 3)

        a16_ref[:, :] = a_ref[:, :].astype(jnp.bfloat16)
        b16_ref[:, :] = b_ref[:, :].astype(jnp.bfloat16)

        rdmas = []
        for off in (2, 1, 3):
            tgt = (my + off) % N_DEV
            slot = off - 1
            send_ref[slot] = jnp.dot(
                a16_ref[pl.ds(tgt * m_out, m_out), :], b16_ref[:, :],
                preferred_element_type=jnp.float32,
            ).astype(jnp.bfloat16)
            rdma = pltpu.make_async_remote_copy(
                src_ref=send_ref.at[slot],
                dst_ref=recv_ref.at[3 - off],
                send_sem=send_sems.at[slot],
                recv_sem=recv_sems.at[3 - off],
                device_id=(tgt,),
                device_id_type=pl.DeviceIdType.MESH,
            )
            rdma.start()
            rdmas.append(rdma)

        own = jnp.dot(
            a16_ref[pl.ds(my * m_out, m_out), :], b16_ref[:, :],
            preferred_element_type=jnp.float32,
        )

        for rdma in rdmas:
            rdma.wait()

        out_ref[:, :] = own + (
            recv_ref[0].astype(jnp.float32)
            + recv_ref[1].astype(jnp.float32)
            + recv_ref[2].astype(jnp.float32)
        )

        @functools.partial(pl.run_scoped, sem=pltpu.SemaphoreType.REGULAR)
        def _(sem):
            for off in (1, 2, 3):
                pl.semaphore_signal(
                    sem, inc=1,
                    device_id=((my + off) % N_DEV,),
                    device_id_type=pl.DeviceIdType.MESH,
                )
            pl.semaphore_wait(sem, 3)

    return pl.pallas_call(
        body,
        out_shape=jax.ShapeDtypeStruct((m_out, n), jnp.float32),
        in_specs=[
            pl.BlockSpec(memory_space=pltpu.VMEM),
            pl.BlockSpec(memory_space=pltpu.VMEM),
        ],
        out_specs=pl.BlockSpec(memory_space=pltpu.VMEM),
        scratch_shapes=[
            pltpu.VMEM((N_DEV - 1, m_out, n), jnp.bfloat16),
            pltpu.VMEM((N_DEV - 1, m_out, n), jnp.bfloat16),
            pltpu.VMEM((m, k), jnp.bfloat16),
            pltpu.VMEM((k, n), jnp.bfloat16),
            pltpu.SemaphoreType.DMA((N_DEV - 1,)),
            pltpu.SemaphoreType.DMA((N_DEV - 1,)),
        ],
        compiler_params=pltpu.CompilerParams(collective_id=0),
    )(A, B)
